# baseline (speedup 1.0000x reference)
import functools

import jax
import jax.numpy as jnp
from jax import lax
from jax.experimental import pallas as pl
from jax.experimental.pallas import tpu as pltpu

H, W = 28, 28
KH, KW = 3, 3
OH, OW = H - KH + 1, W - KW + 1
C_OUT = 32
OC = OW * C_OUT
RK = KH * W
N_CLASSES = 10
TN = 512


def _net_kernel(x_ref, w2_ref, bc_ref, wl_ref, bl_ref, out_ref, xs):
    tn = out_ref.shape[1 - 1]
    xs[...] = x_ref[:, 0, :, :].reshape(tn, H * W).astype(jnp.bfloat16)
    acc = jnp.zeros((tn, N_CLASSES), jnp.float32)
    for oi in range(OH):
        pat = xs[:, oi * W:oi * W + RK]
        a = jnp.dot(pat, w2_ref[...],
                    preferred_element_type=jnp.float32)
        ab = jnp.maximum(a + bc_ref[...], 0.0).astype(jnp.bfloat16)
        acc = acc + lax.dot_general(
            ab, wl_ref[oi],
            dimension_numbers=(((1,), (1,)), ((), ())),
            preferred_element_type=jnp.float32)
    out_ref[...] = acc + bl_ref[...]


@functools.partial(jax.jit, static_argnames=("tn",))
def _forward(x, w_conv, b_conv, w_lin, b_lin, *, tn=TN):
    n = x.shape[0]
    tn = min(tn, max(8, pl.cdiv(n, 8) * 8))
    n_tiles = pl.cdiv(n, tn)
    n_pad = n_tiles * tn

    xf = x
    if n_pad != n:
        xf = jnp.pad(xf, ((0, n_pad - n), (0, 0), (0, 0), (0, 0)))

    wt = jnp.transpose(w_conv[:, 0, :, :], (1, 2, 0))
    eye = jnp.stack([jnp.eye(W, OW, k=-dj, dtype=w_conv.dtype)
                     for dj in range(KW)])
    w2 = jnp.einsum("jko,djc->dkoc", eye, wt).reshape(RK, OC)
    w2 = w2.astype(jnp.bfloat16)
    bc = jnp.tile(b_conv.astype(jnp.float32), OW).reshape(1, OC)

    wl = (w_lin.reshape(N_CLASSES, C_OUT, OH, OW)
               .transpose(2, 0, 3, 1)
               .reshape(OH, N_CLASSES, OC)).astype(jnp.bfloat16)
    bl = b_lin.reshape(1, N_CLASSES).astype(jnp.float32)

    out = pl.pallas_call(
        _net_kernel,
        out_shape=jax.ShapeDtypeStruct((n_pad, N_CLASSES), jnp.float32),
        grid=(n_tiles,),
        in_specs=[
            pl.BlockSpec((tn, 1, H, W), lambda i: (i, 0, 0, 0)),
            pl.BlockSpec((RK, OC), lambda i: (0, 0)),
            pl.BlockSpec((1, OC), lambda i: (0, 0)),
            pl.BlockSpec((OH, N_CLASSES, OC), lambda i: (0, 0, 0)),
            pl.BlockSpec((1, N_CLASSES), lambda i: (0, 0)),
        ],
        out_specs=pl.BlockSpec((tn, N_CLASSES), lambda i: (i, 0)),
        scratch_shapes=[pltpu.VMEM((tn, H * W), jnp.bfloat16)],
        compiler_params=pltpu.CompilerParams(
            dimension_semantics=("parallel",),
            vmem_limit_bytes=64 << 20),
    )(xf, w2, bc, wl, bl)
    return out[:n]


def kernel(x, w_conv, b_conv, w_lin, b_lin):
    return _forward(x, w_conv, b_conv, w_lin, b_lin)


# speedup vs baseline: 1.3841x; 1.3841x over previous
import functools

import jax
import jax.numpy as jnp
from jax.experimental import pallas as pl
from jax.experimental.pallas import tpu as pltpu

H, W = 28, 28
KH, KW = 3, 3
OH, OW = H - KH + 1, W - KW + 1
C_OUT = 32
OC = OW * C_OUT
RK = KH * W
N_CLASSES = 10
BN = 512


def _net_kernel(x_ref, wr_ref, bc_ref, wl_ref, bl_ref, out_ref):
    bn = out_ref.shape[1]
    acc = jnp.zeros((N_CLASSES, bn), jnp.float32)
    for oi in range(OH):
        pat = x_ref[oi * W:oi * W + RK, :].astype(jnp.bfloat16)
        a = jnp.dot(wr_ref[...], pat,
                    preferred_element_type=jnp.float32)
        ab = jnp.maximum(a + bc_ref[...], 0.0).astype(jnp.bfloat16)
        acc = acc + jnp.dot(wl_ref[oi], ab,
                            preferred_element_type=jnp.float32)
    out_ref[...] = acc + bl_ref[...]


@functools.partial(jax.jit, static_argnames=("bn",))
def _forward(x, w_conv, b_conv, w_lin, b_lin, *, bn=BN):
    n = x.shape[0]
    xt = jnp.transpose(x, (2, 3, 1, 0)).reshape(H * W, n)
    bn = min(bn, max(128, pl.cdiv(n, 128) * 128))
    n_tiles = pl.cdiv(n, bn)
    n_pad = n_tiles * bn
    if n_pad != n:
        xt = jnp.pad(xt, ((0, 0), (0, n_pad - n)))

    wt = jnp.transpose(w_conv[:, 0, :, :], (1, 2, 0))
    eye = jnp.stack([jnp.eye(W, OW, k=-dj, dtype=w_conv.dtype)
                     for dj in range(KW)])
    wr = (jnp.einsum("jko,djc->ocdk", eye, wt)
             .reshape(OC, RK).astype(jnp.bfloat16))
    bc = jnp.tile(b_conv.astype(jnp.float32), OW).reshape(OC, 1)

    wl = (w_lin.reshape(N_CLASSES, C_OUT, OH, OW)
               .transpose(2, 0, 3, 1)
               .reshape(OH, N_CLASSES, OC)).astype(jnp.bfloat16)
    bl = b_lin.reshape(N_CLASSES, 1).astype(jnp.float32)

    out = pl.pallas_call(
        _net_kernel,
        out_shape=jax.ShapeDtypeStruct((N_CLASSES, n_pad), jnp.float32),
        grid=(n_tiles,),
        in_specs=[
            pl.BlockSpec((H * W, bn), lambda i: (0, i)),
            pl.BlockSpec((OC, RK), lambda i: (0, 0)),
            pl.BlockSpec((OC, 1), lambda i: (0, 0)),
            pl.BlockSpec((OH, N_CLASSES, OC), lambda i: (0, 0, 0)),
            pl.BlockSpec((N_CLASSES, 1), lambda i: (0, 0)),
        ],
        out_specs=pl.BlockSpec((N_CLASSES, bn), lambda i: (0, i)),
        compiler_params=pltpu.CompilerParams(
            dimension_semantics=("parallel",),
            vmem_limit_bytes=64 << 20),
    )(xt, wr, bc, wl, bl)
    return out[:, :n].T


def kernel(x, w_conv, b_conv, w_lin, b_lin):
    return _forward(x, w_conv, b_conv, w_lin, b_lin)


# speedup vs baseline: 2.3588x; 1.7042x over previous
import functools

import jax
import jax.numpy as jnp
from jax.experimental import pallas as pl
from jax.experimental.pallas import tpu as pltpu

H, W = 28, 28
KH, KW = 3, 3
OH, OW = H - KH + 1, W - KW + 1
C_OUT = 32
OC = OW * C_OUT
RK = KH * W
N_CLASSES = 10
BN = 1024


def _net_kernel(x_ref, wr_ref, bc_ref, wl_ref, bl_ref, out_ref, xs):
    bn = out_ref.shape[1]
    xs[...] = x_ref[...].reshape(H * W, bn).astype(jnp.bfloat16)
    acc = jnp.zeros((N_CLASSES, bn), jnp.float32)
    for oi in range(OH):
        pat = xs[oi * W:oi * W + RK, :]
        a = jnp.dot(wr_ref[...], pat,
                    preferred_element_type=jnp.float32)
        ab = jnp.maximum(a + bc_ref[...], 0.0).astype(jnp.bfloat16)
        acc = acc + jnp.dot(wl_ref[oi], ab,
                            preferred_element_type=jnp.float32)
    out_ref[...] = acc + bl_ref[...]


@functools.partial(jax.jit, static_argnames=("bn",))
def _forward(x, w_conv, b_conv, w_lin, b_lin, *, bn=BN):
    n = x.shape[0]
    bn = min(bn, max(128, pl.cdiv(n, 128) * 128))
    n_tiles = pl.cdiv(n, bn)
    n_pad = n_tiles * bn
    if n_pad == n:
        xt = jnp.transpose(x, (2, 3, 1, 0)).reshape(H * W, n // 128, 128)
    else:
        xt = jnp.pad(jnp.transpose(x, (2, 3, 1, 0)).reshape(H * W, n),
                     ((0, 0), (0, n_pad - n))).reshape(H * W, n_pad // 128, 128)

    wt = jnp.transpose(w_conv[:, 0, :, :], (1, 2, 0))
    eye = jnp.stack([jnp.eye(W, OW, k=-dj, dtype=w_conv.dtype)
                     for dj in range(KW)])
    wr = (jnp.einsum("jko,djc->ocdk", eye, wt)
             .reshape(OC, RK).astype(jnp.bfloat16))
    bc = jnp.tile(b_conv.astype(jnp.float32), OW).reshape(OC, 1)

    wl = (w_lin.reshape(N_CLASSES, C_OUT, OH, OW)
               .transpose(2, 0, 3, 1)
               .reshape(OH, N_CLASSES, OC)).astype(jnp.bfloat16)
    bl = b_lin.reshape(N_CLASSES, 1).astype(jnp.float32)

    out = pl.pallas_call(
        _net_kernel,
        out_shape=jax.ShapeDtypeStruct((N_CLASSES, n_pad), jnp.float32),
        grid=(n_tiles,),
        in_specs=[
            pl.BlockSpec((H * W, bn // 128, 128), lambda i: (0, i, 0)),
            pl.BlockSpec((OC, RK), lambda i: (0, 0)),
            pl.BlockSpec((OC, 1), lambda i: (0, 0)),
            pl.BlockSpec((OH, N_CLASSES, OC), lambda i: (0, 0, 0)),
            pl.BlockSpec((N_CLASSES, 1), lambda i: (0, 0)),
        ],
        out_specs=pl.BlockSpec((N_CLASSES, bn), lambda i: (0, i)),
        scratch_shapes=[pltpu.VMEM((H * W, bn), jnp.bfloat16)],
        compiler_params=pltpu.CompilerParams(
            dimension_semantics=("parallel",),
            vmem_limit_bytes=64 << 20),
    )(xt, wr, bc, wl, bl)
    return out[:, :n].T


def kernel(x, w_conv, b_conv, w_lin, b_lin):
    return _forward(x, w_conv, b_conv, w_lin, b_lin)
